# baseline (speedup 1.0000x reference)
import functools

import jax
import jax.numpy as jnp
from jax import lax
from jax.experimental import pallas as pl
from jax.experimental.pallas import tpu as pltpu
from jax.experimental.pallas import tpu_sc as plsc

N_LEVELS = 16
F = 8
N_FREQ = 12
NUM_LODS = 8
N_NEURONS = 64
BATCH = 262144

PRIME_I32 = -1640531535
HASH_MASK = (1 << 19) - 1
OFF_BASE = 349440 - 6 * 524288

NC = 2
NS = 16
NW = NC * NS
BW = BATCH // NW
CHUNK = 256
GROUPS = CHUNK // 16
NROW = CHUNK * 8
NIDX = NROW // 128


def _scale_for(lev):
    return lax.bitcast_convert_type((lev + 127) << 23, jnp.float32) * 16.0 - 1.0


def _lod_to_level(ll):
    clipped = jnp.minimum(ll * float(NUM_LODS - 1), float(N_LEVELS - 1))
    start = ((float(N_LEVELS - 1) - clipped) * float(F)).astype(jnp.int32)
    return start >> 3, start & 7


def _sc_body(u_hbm, v_hbm, l_hbm, table_hbm, out_hbm,
             u_v, v_v, l_v, idx_v, w_v, rows_v, samp_v, sem):
    wid = lax.axis_index("s") * NC + lax.axis_index("c")
    wbase = wid * BW
    lane = lax.iota(jnp.int32, 16)

    def chunk_body(ci, carry):
        base = wbase + ci * CHUNK
        pltpu.sync_copy(u_hbm.at[pl.ds(base, CHUNK)], u_v)
        pltpu.sync_copy(v_hbm.at[pl.ds(base, CHUNK)], v_v)
        pltpu.sync_copy(l_hbm.at[pl.ds(base, CHUNK)], l_v)

        for g in range(GROUPS):
            s0 = g * 16
            uu = u_v[pl.ds(s0, 16)]
            vv = v_v[pl.ds(s0, 16)]
            ll = l_v[pl.ds(s0, 16)]
            lev0, _ = _lod_to_level(ll)
            for lv in (0, 1):
                lev = jnp.minimum(lev0 + lv, N_LEVELS - 1)
                scale = _scale_for(lev)
                off = (lev << 19) + OFF_BASE
                px = uu * scale + 0.5
                py = vv * scale + 0.5
                gx = px.astype(jnp.int32)
                gy = py.astype(jnp.int32)
                fx = px - gx.astype(jnp.float32)
                fy = py - gy.astype(jnp.float32)
                for k in range(4):
                    dx, dy = k >> 1, k & 1
                    idx = (((gx + dx) ^ ((gy + dy) * PRIME_I32)) & HASH_MASK) + off
                    c = lv * 4 + k
                    idx_v[2 * c + (s0 // 128), pl.ds(s0 % 128, 16)] = idx
                    wx = fx if dx == 1 else 1.0 - fx
                    wy = fy if dy == 1 else 1.0 - fy
                    w_v[c, pl.ds(s0, 16)] = wx * wy

        copies = []
        for j in range(NIDX):
            copies.append(
                pltpu.async_copy(
                    table_hbm.at[idx_v.at[j]], rows_v.at[pl.ds(j * 128, 128)], sem
                )
            )
        for cp in copies:
            cp.wait()

        cbase = ci * CHUNK
        for g in range(GROUPS):
            s0 = g * 16
            ll = l_v[pl.ds(s0, 16)]
            _, o = _lod_to_level(ll)
            ws = [w_v[c, pl.ds(s0, 16)] for c in range(8)]
            svec = lane + s0
            for j in range(F):
                jj = o + j
                lvsel = jj >> 3
                fj = jj & 7
                m0 = lvsel == 0
                rbase = svec + lvsel * (4 * CHUNK)
                acc = None
                for k in range(4):
                    row = rbase + k * CHUNK
                    gval = plsc.load_gather(rows_v, [row, fj])
                    wsel = jnp.where(m0, ws[k], ws[4 + k])
                    term = wsel * gval
                    acc = term if acc is None else acc + term
                samp_v[j, pl.ds(cbase + s0, 16)] = acc
        return carry

    lax.fori_loop(0, BW // CHUNK, chunk_body, 0)
    for j in range(F):
        pltpu.sync_copy(samp_v.at[j], out_hbm.at[j, pl.ds(wbase, BW)])


def _sc_sample(u, v, l, table):
    mesh = plsc.VectorSubcoreMesh(core_axis_name="c", subcore_axis_name="s")
    fn = functools.partial(
        pl.kernel,
        out_type=jax.ShapeDtypeStruct((F, BATCH), jnp.float32),
        mesh=mesh,
        compiler_params=pltpu.CompilerParams(
            use_tc_tiling_on_sc=False, needs_layout_passes=False
        ),
        scratch_types=[
            pltpu.VMEM((CHUNK,), jnp.float32),
            pltpu.VMEM((CHUNK,), jnp.float32),
            pltpu.VMEM((CHUNK,), jnp.float32),
            pltpu.VMEM((NIDX, 128), jnp.int32),
            pltpu.VMEM((8, CHUNK), jnp.float32),
            pltpu.VMEM((NROW, F), jnp.float32),
            pltpu.VMEM((F, BW), jnp.float32),
            pltpu.SemaphoreType.DMA,
        ],
    )(_sc_body)
    return fn(u, v, l, table)


def _transpose_body(tt_ref, o_ref):
    o_ref[...] = tt_ref[...].T


def _tc_transpose_table(table_t):
    n = table_t.shape[1]
    bn = 8192
    return pl.pallas_call(
        _transpose_body,
        grid=(pl.cdiv(n, bn),),
        in_specs=[pl.BlockSpec((F, bn), lambda i: (0, i))],
        out_specs=pl.BlockSpec((bn, F), lambda i: (i, 0)),
        out_shape=jax.ShapeDtypeStruct((n, F), jnp.float32),
        compiler_params=pltpu.CompilerParams(
            dimension_semantics=("parallel",),
        ),
    )(table_t)


def _tc_body(xt_ref, samp_ref, a1_ref, a2_ref, b_ref, c_ref, wh_ref, wo_ref, o_ref):
    u = xt_ref[0:1, :]
    v = xt_ref[1:2, :]
    ll = xt_ref[2:3, :]

    fi = lax.broadcasted_iota(jnp.int32, (N_FREQ, 1), 0)
    freqs = lax.bitcast_convert_type((fi + 126) << 23, jnp.float32)
    xu = freqs * u
    xv = freqs * v
    pe_u = jnp.abs(xu - jnp.floor(xu) - 0.5) * 4.0 - 1.0
    pe_v = jnp.abs(xv - jnp.floor(xv) - 0.5) * 4.0 - 1.0

    h = (
        jnp.dot(a1_ref[...], pe_u, preferred_element_type=jnp.float32)
        + jnp.dot(a2_ref[...], pe_v, preferred_element_type=jnp.float32)
        + jnp.dot(b_ref[...], samp_ref[...], preferred_element_type=jnp.float32)
        + c_ref[...] * ll
    )
    h = jnp.where(h >= 0, h, 0.01 * h)
    h = jnp.dot(wh_ref[...], h, preferred_element_type=jnp.float32)
    h = jnp.where(h >= 0, h, 0.01 * h)
    o_ref[...] = jnp.dot(wo_ref[...], h, preferred_element_type=jnp.float32)


def _tc_mlp(xt, samp, W_in, W_h, W_out):
    bn = 4096
    a1 = W_in[0:N_FREQ].T
    a2 = W_in[N_FREQ:2 * N_FREQ].T
    b = W_in[2 * N_FREQ:2 * N_FREQ + F].T
    c = W_in[2 * N_FREQ + F:2 * N_FREQ + F + 1].T
    wo = jnp.zeros((8, N_NEURONS), jnp.float32).at[:3, :].set(W_out.T)
    full = lambda shape: pl.BlockSpec(shape, lambda i: (0, 0))
    out = pl.pallas_call(
        _tc_body,
        grid=(BATCH // bn,),
        in_specs=[
            pl.BlockSpec((3, bn), lambda i: (0, i)),
            pl.BlockSpec((F, bn), lambda i: (0, i)),
            full((N_NEURONS, N_FREQ)),
            full((N_NEURONS, N_FREQ)),
            full((N_NEURONS, F)),
            full((N_NEURONS, 1)),
            full((N_NEURONS, N_NEURONS)),
            full((8, N_NEURONS)),
        ],
        out_specs=pl.BlockSpec((8, bn), lambda i: (0, i)),
        out_shape=jax.ShapeDtypeStruct((8, BATCH), jnp.float32),
        compiler_params=pltpu.CompilerParams(
            dimension_semantics=("parallel",),
        ),
    )(xt, samp, a1, a2, b, c, W_h.T, wo)
    return out[:3].T


@jax.jit
def kernel(x, table, W_in, W_h, W_out):
    xt = x.T
    table_rm = _tc_transpose_table(table.T)
    samp = _sc_sample(xt[0], xt[1], xt[2], table_rm)
    return _tc_mlp(xt, samp, W_in, W_h, W_out)


# speedup vs baseline: 5.4798x; 5.4798x over previous
import functools

import jax
import jax.numpy as jnp
from jax import lax
from jax.experimental import pallas as pl
from jax.experimental.pallas import tpu as pltpu
from jax.experimental.pallas import tpu_sc as plsc

N_LEVELS = 16
F = 8
N_FREQ = 12
NUM_LODS = 8
N_NEURONS = 64
BATCH = 262144
TROWS = 5592320

PRIME_I32 = -1640531535
HASH_MASK = (1 << 19) - 1
OFF_BASE = 349440 - 6 * 524288

NC = 2
NS = 16
NW = NC * NS
BW = BATCH // NW
CHUNK = 128
GROUPS = CHUNK // 16
NWORD = CHUNK * 32
NIDX = NWORD // 128


def _scale_for(lev):
    return lax.bitcast_convert_type((lev + 127) << 23, jnp.float32) * 16.0 - 1.0


def _lod_to_level(ll):
    clipped = jnp.minimum(ll * float(NUM_LODS - 1), float(N_LEVELS - 1))
    start = ((float(N_LEVELS - 1) - clipped) * float(F)).astype(jnp.int32)
    return start >> 3, start & 7


def _sc_body(u_hbm, v_hbm, l_hbm, t1d_hbm, out_hbm,
             u_v, v_v, l_v, idx_v, w_v, words_v, samp_v, sem):
    wid = lax.axis_index("s") * NC + lax.axis_index("c")
    wbase = wid * BW

    def chunk_body(ci, carry):
        base = wbase + ci * CHUNK
        pltpu.sync_copy(u_hbm.at[pl.ds(base, CHUNK)], u_v)
        pltpu.sync_copy(v_hbm.at[pl.ds(base, CHUNK)], v_v)
        pltpu.sync_copy(l_hbm.at[pl.ds(base, CHUNK)], l_v)

        def pass_a(g, cr):
            s0 = g * 16
            uu = u_v[pl.ds(s0, 16)]
            vv = v_v[pl.ds(s0, 16)]
            ll = l_v[pl.ds(s0, 16)]
            lev0, o = _lod_to_level(ll)
            rows = []
            for lv in (0, 1):
                lev = jnp.minimum(lev0 + lv, N_LEVELS - 1)
                scale = _scale_for(lev)
                off = (lev << 19) + OFF_BASE
                px = uu * scale + 0.5
                py = vv * scale + 0.5
                gx = px.astype(jnp.int32)
                gy = py.astype(jnp.int32)
                fx = px - gx.astype(jnp.float32)
                fy = py - gy.astype(jnp.float32)
                for k in range(4):
                    dx, dy = k >> 1, k & 1
                    r = (((gx + dx) ^ ((gy + dy) * PRIME_I32)) & HASH_MASK) + off
                    rows.append(r)
                    wx = fx if dx == 1 else 1.0 - fx
                    wy = fy if dy == 1 else 1.0 - fy
                    w_v[lv * 4 + k, pl.ds(s0, 16)] = wx * wy
            for j in range(F):
                jj = o + j
                lvsel = jj >> 3
                m0 = lvsel == 0
                fterm = (jj & 7) << 7
                for k in range(4):
                    rsel = jnp.where(m0, rows[k], rows[4 + k])
                    word = ((rsel >> 7) << 10) + (rsel & 127) + fterm
                    flat = (j * 4 + k) * CHUNK + s0
                    idx_v[flat // 128, pl.ds(flat % 128, 16)] = word
            return cr

        lax.fori_loop(0, GROUPS, pass_a, 0)

        copies = []
        for j in range(NIDX):
            copies.append(
                pltpu.async_copy(
                    t1d_hbm.at[idx_v.at[j]], words_v.at[pl.ds(j * 128, 128)], sem
                )
            )
        for cp in copies:
            cp.wait()

        cbase = ci * CHUNK

        def pass_b(g, cr):
            s0 = g * 16
            ll = l_v[pl.ds(s0, 16)]
            _, o = _lod_to_level(ll)
            ws = [w_v[c, pl.ds(s0, 16)] for c in range(8)]
            for j in range(F):
                m0 = ((o + j) >> 3) == 0
                acc = None
                for k in range(4):
                    val = words_v[pl.ds((j * 4 + k) * CHUNK + s0, 16)]
                    wsel = jnp.where(m0, ws[k], ws[4 + k])
                    term = wsel * val
                    acc = term if acc is None else acc + term
                samp_v[j, pl.ds(cbase + s0, 16)] = acc
            return cr

        lax.fori_loop(0, GROUPS, pass_b, 0)
        return carry

    lax.fori_loop(0, BW // CHUNK, chunk_body, 0)
    for j in range(F):
        pltpu.sync_copy(samp_v.at[j], out_hbm.at[j, pl.ds(wbase, BW)])


def _sc_sample(u, v, l, t1d):
    mesh = plsc.VectorSubcoreMesh(core_axis_name="c", subcore_axis_name="s")
    fn = functools.partial(
        pl.kernel,
        out_type=jax.ShapeDtypeStruct((F, BATCH), jnp.float32),
        mesh=mesh,
        compiler_params=pltpu.CompilerParams(
            use_tc_tiling_on_sc=False, needs_layout_passes=False
        ),
        scratch_types=[
            pltpu.VMEM((CHUNK,), jnp.float32),
            pltpu.VMEM((CHUNK,), jnp.float32),
            pltpu.VMEM((CHUNK,), jnp.float32),
            pltpu.VMEM((NIDX, 128), jnp.int32),
            pltpu.VMEM((8, CHUNK), jnp.float32),
            pltpu.VMEM((NWORD,), jnp.float32),
            pltpu.VMEM((F, BW), jnp.float32),
            pltpu.SemaphoreType.DMA,
        ],
    )(_sc_body)
    return fn(u, v, l, t1d)


def _tc_body(xt_ref, samp_ref, a1_ref, a2_ref, b_ref, c_ref, wh_ref, wo_ref, o_ref):
    u = xt_ref[0:1, :]
    v = xt_ref[1:2, :]
    ll = xt_ref[2:3, :]

    fi = lax.broadcasted_iota(jnp.int32, (N_FREQ, 1), 0)
    freqs = lax.bitcast_convert_type((fi + 126) << 23, jnp.float32)
    xu = freqs * u
    xv = freqs * v
    pe_u = jnp.abs(xu - jnp.floor(xu) - 0.5) * 4.0 - 1.0
    pe_v = jnp.abs(xv - jnp.floor(xv) - 0.5) * 4.0 - 1.0

    h = (
        jnp.dot(a1_ref[...], pe_u, preferred_element_type=jnp.float32)
        + jnp.dot(a2_ref[...], pe_v, preferred_element_type=jnp.float32)
        + jnp.dot(b_ref[...], samp_ref[...], preferred_element_type=jnp.float32)
        + c_ref[...] * ll
    )
    h = jnp.where(h >= 0, h, 0.01 * h)
    h = jnp.dot(wh_ref[...], h, preferred_element_type=jnp.float32)
    h = jnp.where(h >= 0, h, 0.01 * h)
    o_ref[...] = jnp.dot(wo_ref[...], h, preferred_element_type=jnp.float32)


def _tc_mlp(xt, samp, W_in, W_h, W_out):
    bn = 4096
    a1 = W_in[0:N_FREQ].T
    a2 = W_in[N_FREQ:2 * N_FREQ].T
    b = W_in[2 * N_FREQ:2 * N_FREQ + F].T
    c = W_in[2 * N_FREQ + F:2 * N_FREQ + F + 1].T
    wo = jnp.zeros((8, N_NEURONS), jnp.float32).at[:3, :].set(W_out.T)
    full = lambda shape: pl.BlockSpec(shape, lambda i: (0, 0))
    out = pl.pallas_call(
        _tc_body,
        grid=(BATCH // bn,),
        in_specs=[
            pl.BlockSpec((3, bn), lambda i: (0, i)),
            pl.BlockSpec((F, bn), lambda i: (0, i)),
            full((N_NEURONS, N_FREQ)),
            full((N_NEURONS, N_FREQ)),
            full((N_NEURONS, F)),
            full((N_NEURONS, 1)),
            full((N_NEURONS, N_NEURONS)),
            full((8, N_NEURONS)),
        ],
        out_specs=pl.BlockSpec((8, bn), lambda i: (0, i)),
        out_shape=jax.ShapeDtypeStruct((8, BATCH), jnp.float32),
        compiler_params=pltpu.CompilerParams(
            dimension_semantics=("parallel",),
        ),
    )(xt, samp, a1, a2, b, c, W_h.T, wo)
    return out[:3].T


@jax.jit
def kernel(x, table, W_in, W_h, W_out):
    xt = x.T
    t1d = table.T.reshape(F, TROWS // 128, 128).transpose(1, 0, 2).reshape(-1)
    samp = _sc_sample(xt[0], xt[1], xt[2], t1d)
    return _tc_mlp(xt, samp, W_in, W_h, W_out)


# speedup vs baseline: 7.7476x; 1.4138x over previous
import functools

import jax
import jax.numpy as jnp
from jax import lax
from jax.experimental import pallas as pl
from jax.experimental.pallas import tpu as pltpu
from jax.experimental.pallas import tpu_sc as plsc

N_LEVELS = 16
F = 8
N_FREQ = 12
NUM_LODS = 8
N_NEURONS = 64
BATCH = 262144
TROWS = 5592320

PRIME_I32 = -1640531535
HASH_MASK = (1 << 19) - 1
OFF_BASE = 349440 - 6 * 524288

NC = 2
NS = 16
NW = NC * NS
BW = BATCH // NW
CHUNK = 128
GROUPS = CHUNK // 16
NWORD = CHUNK * 32
NIDX = NWORD // 128


def _scale_for(lev):
    return lax.bitcast_convert_type((lev + 127) << 23, jnp.float32) * 16.0 - 1.0


def _lod_to_level(ll):
    clipped = jnp.minimum(ll * float(NUM_LODS - 1), float(N_LEVELS - 1))
    start = ((float(N_LEVELS - 1) - clipped) * float(F)).astype(jnp.int32)
    return start >> 3, start & 7


def _sc_body(u_hbm, v_hbm, l_hbm, t1d_hbm, out_hbm,
             u_v, v_v, l_v, idx0, idx1, w0, w1, words0, words1, samp_v,
             sem0, sem1):
    wid = lax.axis_index("s") * NC + lax.axis_index("c")
    wbase = wid * BW
    pltpu.sync_copy(u_hbm.at[pl.ds(wbase, BW)], u_v)
    pltpu.sync_copy(v_hbm.at[pl.ds(wbase, BW)], v_v)
    pltpu.sync_copy(l_hbm.at[pl.ds(wbase, BW)], l_v)

    def pass_a(ci, idx_v, w_v):
        def body(g, cr):
            s0 = ci * CHUNK + g * 16
            uu = u_v[pl.ds(s0, 16)]
            vv = v_v[pl.ds(s0, 16)]
            ll = l_v[pl.ds(s0, 16)]
            lev0, o = _lod_to_level(ll)
            rows = []
            for lv in (0, 1):
                lev = jnp.minimum(lev0 + lv, N_LEVELS - 1)
                scale = _scale_for(lev)
                off = (lev << 19) + OFF_BASE
                px = uu * scale + 0.5
                py = vv * scale + 0.5
                gx = px.astype(jnp.int32)
                gy = py.astype(jnp.int32)
                fx = px - gx.astype(jnp.float32)
                fy = py - gy.astype(jnp.float32)
                for k in range(4):
                    dx, dy = k >> 1, k & 1
                    r = (((gx + dx) ^ ((gy + dy) * PRIME_I32)) & HASH_MASK) + off
                    rows.append(r)
                    wx = fx if dx == 1 else 1.0 - fx
                    wy = fy if dy == 1 else 1.0 - fy
                    w_v[lv * 4 + k, pl.ds(g * 16, 16)] = wx * wy
            for j in range(F):
                jj = o + j
                m0 = (jj >> 3) == 0
                fterm = (jj & 7) << 7
                for k in range(4):
                    rsel = jnp.where(m0, rows[k], rows[4 + k])
                    word = ((rsel >> 7) << 10) + (rsel & 127) + fterm
                    flat = (j * 4 + k) * CHUNK + g * 16
                    idx_v[flat // 128, pl.ds(flat % 128, 16)] = word
            return cr

        lax.fori_loop(0, GROUPS, body, 0)

    def fire(idx_v, words_v, sem):
        def body(j, cr):
            pltpu.async_copy(
                t1d_hbm.at[idx_v.at[j]], words_v.at[pl.ds(j * 128, 128)], sem
            )
            return cr

        lax.fori_loop(0, NIDX, body, 0)

    def drain(words_v, sem):
        pltpu.make_async_copy(t1d_hbm.at[pl.ds(0, NWORD)], words_v, sem).wait()

    def pass_b(ci, words_v, w_v):
        def body(g, cr):
            s0 = g * 16
            ll = l_v[pl.ds(ci * CHUNK + s0, 16)]
            _, o = _lod_to_level(ll)
            ws = [w_v[c, pl.ds(s0, 16)] for c in range(8)]
            for j in range(F):
                m0 = ((o + j) >> 3) == 0
                acc = None
                for k in range(4):
                    val = words_v[pl.ds((j * 4 + k) * CHUNK + s0, 16)]
                    wsel = jnp.where(m0, ws[k], ws[4 + k])
                    term = wsel * val
                    acc = term if acc is None else acc + term
                samp_v[j, pl.ds(ci * CHUNK + s0, 16)] = acc
            return cr

        lax.fori_loop(0, GROUPS, body, 0)

    def pair_body(t, carry):
        a = 2 * t
        b = 2 * t + 1
        pass_a(a, idx0, w0)
        fire(idx0, words0, sem0)
        pass_a(b, idx1, w1)
        fire(idx1, words1, sem1)
        drain(words0, sem0)
        pass_b(a, words0, w0)
        drain(words1, sem1)
        pass_b(b, words1, w1)
        return carry

    lax.fori_loop(0, BW // (2 * CHUNK), pair_body, 0)
    for j in range(F):
        pltpu.sync_copy(samp_v.at[j], out_hbm.at[j, pl.ds(wbase, BW)])


def _sc_sample(u, v, l, t1d):
    mesh = plsc.VectorSubcoreMesh(core_axis_name="c", subcore_axis_name="s")
    fn = functools.partial(
        pl.kernel,
        out_type=jax.ShapeDtypeStruct((F, BATCH), jnp.float32),
        mesh=mesh,
        compiler_params=pltpu.CompilerParams(
            use_tc_tiling_on_sc=False, needs_layout_passes=False
        ),
        scratch_types=[
            pltpu.VMEM((BW,), jnp.float32),
            pltpu.VMEM((BW,), jnp.float32),
            pltpu.VMEM((BW,), jnp.float32),
            pltpu.VMEM((NIDX, 128), jnp.int32),
            pltpu.VMEM((NIDX, 128), jnp.int32),
            pltpu.VMEM((8, CHUNK), jnp.float32),
            pltpu.VMEM((8, CHUNK), jnp.float32),
            pltpu.VMEM((NWORD,), jnp.float32),
            pltpu.VMEM((NWORD,), jnp.float32),
            pltpu.VMEM((F, BW), jnp.float32),
            pltpu.SemaphoreType.DMA,
            pltpu.SemaphoreType.DMA,
        ],
    )(_sc_body)
    return fn(u, v, l, t1d)


def _tc_body(xt_ref, samp_ref, a1_ref, a2_ref, b_ref, c_ref, wh_ref, wo_ref, o_ref):
    u = xt_ref[0:1, :]
    v = xt_ref[1:2, :]
    ll = xt_ref[2:3, :]

    fi = lax.broadcasted_iota(jnp.int32, (N_FREQ, 1), 0)
    freqs = lax.bitcast_convert_type((fi + 126) << 23, jnp.float32)
    xu = freqs * u
    xv = freqs * v
    pe_u = jnp.abs(xu - jnp.floor(xu) - 0.5) * 4.0 - 1.0
    pe_v = jnp.abs(xv - jnp.floor(xv) - 0.5) * 4.0 - 1.0

    h = (
        jnp.dot(a1_ref[...], pe_u, preferred_element_type=jnp.float32)
        + jnp.dot(a2_ref[...], pe_v, preferred_element_type=jnp.float32)
        + jnp.dot(b_ref[...], samp_ref[...], preferred_element_type=jnp.float32)
        + c_ref[...] * ll
    )
    h = jnp.where(h >= 0, h, 0.01 * h)
    h = jnp.dot(wh_ref[...], h, preferred_element_type=jnp.float32)
    h = jnp.where(h >= 0, h, 0.01 * h)
    o_ref[...] = jnp.dot(wo_ref[...], h, preferred_element_type=jnp.float32)


def _tc_mlp(xt, samp, W_in, W_h, W_out):
    bn = 4096
    a1 = W_in[0:N_FREQ].T
    a2 = W_in[N_FREQ:2 * N_FREQ].T
    b = W_in[2 * N_FREQ:2 * N_FREQ + F].T
    c = W_in[2 * N_FREQ + F:2 * N_FREQ + F + 1].T
    wo = jnp.zeros((8, N_NEURONS), jnp.float32).at[:3, :].set(W_out.T)
    full = lambda shape: pl.BlockSpec(shape, lambda i: (0, 0))
    out = pl.pallas_call(
        _tc_body,
        grid=(BATCH // bn,),
        in_specs=[
            pl.BlockSpec((3, bn), lambda i: (0, i)),
            pl.BlockSpec((F, bn), lambda i: (0, i)),
            full((N_NEURONS, N_FREQ)),
            full((N_NEURONS, N_FREQ)),
            full((N_NEURONS, F)),
            full((N_NEURONS, 1)),
            full((N_NEURONS, N_NEURONS)),
            full((8, N_NEURONS)),
        ],
        out_specs=pl.BlockSpec((8, bn), lambda i: (0, i)),
        out_shape=jax.ShapeDtypeStruct((8, BATCH), jnp.float32),
        compiler_params=pltpu.CompilerParams(
            dimension_semantics=("parallel",),
        ),
    )(xt, samp, a1, a2, b, c, W_h.T, wo)
    return out[:3].T


@jax.jit
def kernel(x, table, W_in, W_h, W_out):
    xt = x.T
    t1d = table.T.reshape(F, TROWS // 128, 128).transpose(1, 0, 2).reshape(-1)
    samp = _sc_sample(xt[0], xt[1], xt[2], t1d)
    return _tc_mlp(xt, samp, W_in, W_h, W_out)


# speedup vs baseline: 7.7961x; 1.0063x over previous
import functools

import jax
import jax.numpy as jnp
from jax import lax
from jax.experimental import pallas as pl
from jax.experimental.pallas import tpu as pltpu
from jax.experimental.pallas import tpu_sc as plsc

N_LEVELS = 16
F = 8
N_FREQ = 12
NUM_LODS = 8
N_NEURONS = 64
BATCH = 262144
TROWS = 5592320

PRIME_I32 = -1640531535
HASH_MASK = (1 << 19) - 1
OFF_BASE = 349440 - 6 * 524288

NC = 2
NS = 16
NW = NC * NS
BW = BATCH // NW
CHUNK = 128
GROUPS = CHUNK // 16
NWORD = CHUNK * 32
NIDX = NWORD // 128


def _scale_for(lev):
    return lax.bitcast_convert_type((lev + 127) << 23, jnp.float32) * 16.0 - 1.0


def _lod_to_level(ll):
    clipped = jnp.minimum(ll * float(NUM_LODS - 1), float(N_LEVELS - 1))
    start = ((float(N_LEVELS - 1) - clipped) * float(F)).astype(jnp.int32)
    return start >> 3, start & 7


def _sc_body(u_hbm, v_hbm, l_hbm, t1d_hbm, out_hbm,
             u_v, v_v, l_v, idx0, idx1, w0, w1, words0, words1, samp_v,
             sem0, sem1):
    wid = lax.axis_index("s") * NC + lax.axis_index("c")
    wbase = wid * BW
    pltpu.sync_copy(u_hbm.at[pl.ds(wbase, BW)], u_v)
    pltpu.sync_copy(v_hbm.at[pl.ds(wbase, BW)], v_v)
    pltpu.sync_copy(l_hbm.at[pl.ds(wbase, BW)], l_v)

    def pass_a(ci, idx_v, w_v):
        def body(g, cr):
            s0 = ci * CHUNK + g * 16
            uu = u_v[pl.ds(s0, 16)]
            vv = v_v[pl.ds(s0, 16)]
            ll = l_v[pl.ds(s0, 16)]
            lev0, o = _lod_to_level(ll)
            rows = []
            for lv in (0, 1):
                lev = jnp.minimum(lev0 + lv, N_LEVELS - 1)
                scale = _scale_for(lev)
                off = (lev << 19) + OFF_BASE
                px = uu * scale + 0.5
                py = vv * scale + 0.5
                gx = px.astype(jnp.int32)
                gy = py.astype(jnp.int32)
                fx = px - gx.astype(jnp.float32)
                fy = py - gy.astype(jnp.float32)
                for k in range(4):
                    dx, dy = k >> 1, k & 1
                    r = (((gx + dx) ^ ((gy + dy) * PRIME_I32)) & HASH_MASK) + off
                    rows.append(((r >> 7) << 10) + (r & 127))
                    wx = fx if dx == 1 else 1.0 - fx
                    wy = fy if dy == 1 else 1.0 - fy
                    w_v[lv * 4 + k, pl.ds(g * 16, 16)] = wx * wy
            for j in range(F):
                jj = o + j
                m0 = (jj >> 3) == 0
                fterm = (jj & 7) << 7
                for k in range(4):
                    word = jnp.where(m0, rows[k], rows[4 + k]) + fterm
                    flat = (j * 4 + k) * CHUNK + g * 16
                    idx_v[flat // 128, pl.ds(flat % 128, 16)] = word
            return cr

        lax.fori_loop(0, GROUPS, body, 0)

    def fire(idx_v, words_v, sem):
        def body(j, cr):
            pltpu.async_copy(
                t1d_hbm.at[idx_v.at[j]], words_v.at[pl.ds(j * 128, 128)], sem
            )
            return cr

        lax.fori_loop(0, NIDX, body, 0)

    def drain(words_v, sem):
        pltpu.make_async_copy(t1d_hbm.at[pl.ds(0, NWORD)], words_v, sem).wait()

    def pass_b(ci, words_v, w_v):
        def body(g, cr):
            s0 = g * 16
            ll = l_v[pl.ds(ci * CHUNK + s0, 16)]
            _, o = _lod_to_level(ll)
            ws = [w_v[c, pl.ds(s0, 16)] for c in range(8)]
            for j in range(F):
                m0 = ((o + j) >> 3) == 0
                acc = None
                for k in range(4):
                    val = words_v[pl.ds((j * 4 + k) * CHUNK + s0, 16)]
                    wsel = jnp.where(m0, ws[k], ws[4 + k])
                    term = wsel * val
                    acc = term if acc is None else acc + term
                samp_v[j, pl.ds(ci * CHUNK + s0, 16)] = acc
            return cr

        lax.fori_loop(0, GROUPS, body, 0)

    def pair_body(t, carry):
        a = 2 * t
        b = 2 * t + 1
        pass_a(a, idx0, w0)
        fire(idx0, words0, sem0)
        pass_a(b, idx1, w1)
        fire(idx1, words1, sem1)
        drain(words0, sem0)
        pass_b(a, words0, w0)
        drain(words1, sem1)
        pass_b(b, words1, w1)
        return carry

    lax.fori_loop(0, BW // (2 * CHUNK), pair_body, 0)
    for j in range(F):
        pltpu.sync_copy(samp_v.at[j], out_hbm.at[j, pl.ds(wbase, BW)])


def _sc_sample(u, v, l, t1d):
    mesh = plsc.VectorSubcoreMesh(core_axis_name="c", subcore_axis_name="s")
    fn = functools.partial(
        pl.kernel,
        out_type=jax.ShapeDtypeStruct((F, BATCH), jnp.float32),
        mesh=mesh,
        compiler_params=pltpu.CompilerParams(
            use_tc_tiling_on_sc=False, needs_layout_passes=False
        ),
        scratch_types=[
            pltpu.VMEM((BW,), jnp.float32),
            pltpu.VMEM((BW,), jnp.float32),
            pltpu.VMEM((BW,), jnp.float32),
            pltpu.VMEM((NIDX, 128), jnp.int32),
            pltpu.VMEM((NIDX, 128), jnp.int32),
            pltpu.VMEM((8, CHUNK), jnp.float32),
            pltpu.VMEM((8, CHUNK), jnp.float32),
            pltpu.VMEM((NWORD,), jnp.float32),
            pltpu.VMEM((NWORD,), jnp.float32),
            pltpu.VMEM((F, BW), jnp.float32),
            pltpu.SemaphoreType.DMA,
            pltpu.SemaphoreType.DMA,
        ],
    )(_sc_body)
    return fn(u, v, l, t1d)


def _tc_body(xt_ref, samp_ref, a1_ref, a2_ref, b_ref, c_ref, wh_ref, wo_ref, o_ref):
    u = xt_ref[0:1, :]
    v = xt_ref[1:2, :]
    ll = xt_ref[2:3, :]

    fi = lax.broadcasted_iota(jnp.int32, (N_FREQ, 1), 0)
    freqs = lax.bitcast_convert_type((fi + 126) << 23, jnp.float32)
    xu = freqs * u
    xv = freqs * v
    pe_u = jnp.abs(xu - jnp.floor(xu) - 0.5) * 4.0 - 1.0
    pe_v = jnp.abs(xv - jnp.floor(xv) - 0.5) * 4.0 - 1.0

    h = (
        jnp.dot(a1_ref[...], pe_u, preferred_element_type=jnp.float32)
        + jnp.dot(a2_ref[...], pe_v, preferred_element_type=jnp.float32)
        + jnp.dot(b_ref[...], samp_ref[...], preferred_element_type=jnp.float32)
        + c_ref[...] * ll
    )
    h = jnp.where(h >= 0, h, 0.01 * h)
    h = jnp.dot(wh_ref[...], h, preferred_element_type=jnp.float32)
    h = jnp.where(h >= 0, h, 0.01 * h)
    o_ref[...] = jnp.dot(wo_ref[...], h, preferred_element_type=jnp.float32)


def _tc_mlp(xt, samp, W_in, W_h, W_out):
    bn = 4096
    a1 = W_in[0:N_FREQ].T
    a2 = W_in[N_FREQ:2 * N_FREQ].T
    b = W_in[2 * N_FREQ:2 * N_FREQ + F].T
    c = W_in[2 * N_FREQ + F:2 * N_FREQ + F + 1].T
    wo = jnp.zeros((8, N_NEURONS), jnp.float32).at[:3, :].set(W_out.T)
    full = lambda shape: pl.BlockSpec(shape, lambda i: (0, 0))
    out = pl.pallas_call(
        _tc_body,
        grid=(BATCH // bn,),
        in_specs=[
            pl.BlockSpec((3, bn), lambda i: (0, i)),
            pl.BlockSpec((F, bn), lambda i: (0, i)),
            full((N_NEURONS, N_FREQ)),
            full((N_NEURONS, N_FREQ)),
            full((N_NEURONS, F)),
            full((N_NEURONS, 1)),
            full((N_NEURONS, N_NEURONS)),
            full((8, N_NEURONS)),
        ],
        out_specs=pl.BlockSpec((8, bn), lambda i: (0, i)),
        out_shape=jax.ShapeDtypeStruct((8, BATCH), jnp.float32),
        compiler_params=pltpu.CompilerParams(
            dimension_semantics=("parallel",),
        ),
    )(xt, samp, a1, a2, b, c, W_h.T, wo)
    return out[:3].T


@jax.jit
def kernel(x, table, W_in, W_h, W_out):
    xt = x.T
    t1d = table.T.reshape(F, TROWS // 128, 128).transpose(1, 0, 2).reshape(-1)
    samp = _sc_sample(xt[0], xt[1], xt[2], t1d)
    return _tc_mlp(xt, samp, W_in, W_h, W_out)


# speedup vs baseline: 8.1445x; 1.0447x over previous
import functools

import jax
import jax.numpy as jnp
from jax import lax
from jax.experimental import pallas as pl
from jax.experimental.pallas import tpu as pltpu
from jax.experimental.pallas import tpu_sc as plsc

N_LEVELS = 16
F = 8
N_FREQ = 12
NUM_LODS = 8
N_NEURONS = 64
BATCH = 262144
TROWS = 5592320

PRIME_I32 = -1640531535
HASH_MASK = (1 << 19) - 1
OFF_BASE = 349440 - 6 * 524288

NC = 2
NS = 16
NW = NC * NS
BW = BATCH // NW
CHUNK = 128
GROUPS = CHUNK // 16
NWORD = CHUNK * 32
NIDX = NWORD // 128


def _scale_for(lev):
    return lax.bitcast_convert_type((lev + 127) << 23, jnp.float32) * 16.0 - 1.0


def _lod_to_level(ll):
    clipped = jnp.minimum(ll * float(NUM_LODS - 1), float(N_LEVELS - 1))
    start = ((float(N_LEVELS - 1) - clipped) * float(F)).astype(jnp.int32)
    return start >> 3, start & 7


def _sc_body(u_hbm, v_hbm, l_hbm, t1d_hbm, out_hbm,
             u_v, v_v, l_v, idx0, idx1, w0, w1, words0, words1, samp_v,
             sem0, sem1):
    wid = lax.axis_index("s") * NC + lax.axis_index("c")
    wbase = wid * BW
    pltpu.sync_copy(u_hbm.at[pl.ds(wbase, BW)], u_v)
    pltpu.sync_copy(v_hbm.at[pl.ds(wbase, BW)], v_v)
    pltpu.sync_copy(l_hbm.at[pl.ds(wbase, BW)], l_v)

    def pass_a(ci, idx_v, w_v):
        def body(g, cr):
            s0 = ci * CHUNK + g * 16
            uu = u_v[pl.ds(s0, 16)]
            vv = v_v[pl.ds(s0, 16)]
            ll = l_v[pl.ds(s0, 16)]
            lev0, o = _lod_to_level(ll)
            rows = []
            for lv in (0, 1):
                lev = jnp.minimum(lev0 + lv, N_LEVELS - 1)
                scale = _scale_for(lev)
                off = (lev << 19) + OFF_BASE
                px = uu * scale + 0.5
                py = vv * scale + 0.5
                gx = px.astype(jnp.int32)
                gy = py.astype(jnp.int32)
                fx = px - gx.astype(jnp.float32)
                fy = py - gy.astype(jnp.float32)
                for k in range(4):
                    dx, dy = k >> 1, k & 1
                    r = (((gx + dx) ^ ((gy + dy) * PRIME_I32)) & HASH_MASK) + off
                    rows.append(((r >> 7) << 10) + (r & 127))
                    wx = fx if dx == 1 else 1.0 - fx
                    wy = fy if dy == 1 else 1.0 - fy
                    w_v[lv * 4 + k, pl.ds(g * 16, 16)] = wx * wy
            for j in range(F):
                jj = o + j
                m0 = (jj >> 3) == 0
                fterm = (jj & 7) << 7
                for k in range(4):
                    word = jnp.where(m0, rows[k], rows[4 + k]) + fterm
                    flat = (j * 4 + k) * CHUNK + g * 16
                    idx_v[flat // 128, pl.ds(flat % 128, 16)] = word
            return cr

        lax.fori_loop(0, GROUPS, body, 0)

    def fire(idx_v, words_v, sem):
        def body(j, cr):
            pltpu.async_copy(
                t1d_hbm.at[idx_v.at[j]], words_v.at[pl.ds(j * 128, 128)], sem
            )
            return cr

        lax.fori_loop(0, NIDX, body, 0)

    def drain(words_v, sem):
        pltpu.make_async_copy(t1d_hbm.at[pl.ds(0, NWORD)], words_v, sem).wait()

    def pass_b(ci, words_v, w_v):
        def body(g, cr):
            s0 = g * 16
            ll = l_v[pl.ds(ci * CHUNK + s0, 16)]
            _, o = _lod_to_level(ll)
            ws = [w_v[c, pl.ds(s0, 16)] for c in range(8)]
            for j in range(F):
                m0 = ((o + j) >> 3) == 0
                acc = None
                for k in range(4):
                    val = words_v[pl.ds((j * 4 + k) * CHUNK + s0, 16)]
                    wsel = jnp.where(m0, ws[k], ws[4 + k])
                    term = wsel * val
                    acc = term if acc is None else acc + term
                samp_v[j, pl.ds(ci * CHUNK + s0, 16)] = acc
            return cr

        lax.fori_loop(0, GROUPS, body, 0)

    NCH = BW // CHUNK
    pass_a(0, idx0, w0)
    fire(idx0, words0, sem0)

    def pair_body(t, carry):
        a = 2 * t + 1
        b = 2 * t + 2
        pass_a(a, idx1, w1)
        fire(idx1, words1, sem1)
        drain(words0, sem0)
        pass_b(a - 1, words0, w0)
        pass_a(b, idx0, w0)
        fire(idx0, words0, sem0)
        drain(words1, sem1)
        pass_b(a, words1, w1)
        return carry

    lax.fori_loop(0, (NCH - 2) // 2, pair_body, 0)
    pass_a(NCH - 1, idx1, w1)
    fire(idx1, words1, sem1)
    drain(words0, sem0)
    pass_b(NCH - 2, words0, w0)
    drain(words1, sem1)
    pass_b(NCH - 1, words1, w1)
    for j in range(F):
        pltpu.sync_copy(samp_v.at[j], out_hbm.at[j, pl.ds(wbase, BW)])


def _sc_sample(u, v, l, t1d):
    mesh = plsc.VectorSubcoreMesh(core_axis_name="c", subcore_axis_name="s")
    fn = functools.partial(
        pl.kernel,
        out_type=jax.ShapeDtypeStruct((F, BATCH), jnp.float32),
        mesh=mesh,
        compiler_params=pltpu.CompilerParams(
            use_tc_tiling_on_sc=False, needs_layout_passes=False
        ),
        scratch_types=[
            pltpu.VMEM((BW,), jnp.float32),
            pltpu.VMEM((BW,), jnp.float32),
            pltpu.VMEM((BW,), jnp.float32),
            pltpu.VMEM((NIDX, 128), jnp.int32),
            pltpu.VMEM((NIDX, 128), jnp.int32),
            pltpu.VMEM((8, CHUNK), jnp.float32),
            pltpu.VMEM((8, CHUNK), jnp.float32),
            pltpu.VMEM((NWORD,), jnp.float32),
            pltpu.VMEM((NWORD,), jnp.float32),
            pltpu.VMEM((F, BW), jnp.float32),
            pltpu.SemaphoreType.DMA,
            pltpu.SemaphoreType.DMA,
        ],
    )(_sc_body)
    return fn(u, v, l, t1d)


def _tc_body(xt_ref, samp_ref, a1_ref, a2_ref, b_ref, c_ref, wh_ref, wo_ref, o_ref):
    u = xt_ref[0:1, :]
    v = xt_ref[1:2, :]
    ll = xt_ref[2:3, :]

    fi = lax.broadcasted_iota(jnp.int32, (N_FREQ, 1), 0)
    freqs = lax.bitcast_convert_type((fi + 126) << 23, jnp.float32)
    xu = freqs * u
    xv = freqs * v
    pe_u = jnp.abs(xu - jnp.floor(xu) - 0.5) * 4.0 - 1.0
    pe_v = jnp.abs(xv - jnp.floor(xv) - 0.5) * 4.0 - 1.0

    h = (
        jnp.dot(a1_ref[...], pe_u, preferred_element_type=jnp.float32)
        + jnp.dot(a2_ref[...], pe_v, preferred_element_type=jnp.float32)
        + jnp.dot(b_ref[...], samp_ref[...], preferred_element_type=jnp.float32)
        + c_ref[...] * ll
    )
    h = jnp.where(h >= 0, h, 0.01 * h)
    h = jnp.dot(wh_ref[...], h, preferred_element_type=jnp.float32)
    h = jnp.where(h >= 0, h, 0.01 * h)
    o_ref[...] = jnp.dot(wo_ref[...], h, preferred_element_type=jnp.float32)


def _tc_mlp(xt, samp, W_in, W_h, W_out):
    bn = 4096
    a1 = W_in[0:N_FREQ].T
    a2 = W_in[N_FREQ:2 * N_FREQ].T
    b = W_in[2 * N_FREQ:2 * N_FREQ + F].T
    c = W_in[2 * N_FREQ + F:2 * N_FREQ + F + 1].T
    wo = jnp.zeros((8, N_NEURONS), jnp.float32).at[:3, :].set(W_out.T)
    full = lambda shape: pl.BlockSpec(shape, lambda i: (0, 0))
    out = pl.pallas_call(
        _tc_body,
        grid=(BATCH // bn,),
        in_specs=[
            pl.BlockSpec((3, bn), lambda i: (0, i)),
            pl.BlockSpec((F, bn), lambda i: (0, i)),
            full((N_NEURONS, N_FREQ)),
            full((N_NEURONS, N_FREQ)),
            full((N_NEURONS, F)),
            full((N_NEURONS, 1)),
            full((N_NEURONS, N_NEURONS)),
            full((8, N_NEURONS)),
        ],
        out_specs=pl.BlockSpec((8, bn), lambda i: (0, i)),
        out_shape=jax.ShapeDtypeStruct((8, BATCH), jnp.float32),
        compiler_params=pltpu.CompilerParams(
            dimension_semantics=("parallel",),
        ),
    )(xt, samp, a1, a2, b, c, W_h.T, wo)
    return out[:3].T


@jax.jit
def kernel(x, table, W_in, W_h, W_out):
    xt = x.T
    t1d = table.T.reshape(F, TROWS // 128, 128).transpose(1, 0, 2).reshape(-1)
    samp = _sc_sample(xt[0], xt[1], xt[2], t1d)
    return _tc_mlp(xt, samp, W_in, W_h, W_out)
